# baseline (speedup 1.0000x reference)
import functools

import jax
import jax.numpy as jnp
from jax import lax
from jax.experimental import pallas as pl
from jax.experimental.pallas import tpu as pltpu
from jax.experimental.pallas import tpu_sc as plsc

_D = 16
_NW = 32
_CHUNK = 128


@functools.cache
def _build(n_tok: int):
    per_w = n_tok // _NW
    n_ch = per_w // _CHUNK
    mesh = plsc.VectorSubcoreMesh(core_axis_name="c", subcore_axis_name="s")

    @functools.partial(
        pl.kernel,
        mesh=mesh,
        out_type=jax.ShapeDtypeStruct((n_tok, _D), jnp.float32),
        scratch_types=[
            pltpu.VMEM((per_w,), jnp.int32),
            pltpu.VMEM((per_w, _D), jnp.float32),
            pltpu.VMEM((per_w,), jnp.float32),
            pltpu.SemaphoreType.DMA,
            pltpu.SemaphoreType.DMA,
        ],
    )
    def body(idx_hbm, emb_hbm, gates_hbm, out_hbm, idx_v, rows_v, g_v, sem_r, sem_g):
        wid = lax.axis_index("s") * 2 + lax.axis_index("c")
        base = wid * per_w
        pltpu.sync_copy(idx_hbm.at[pl.ds(base, per_w)], idx_v)

        g_copies = []
        for j in range(n_ch):
            g_copies.append(pltpu.async_copy(
                gates_hbm.at[idx_v.at[pl.ds(j * _CHUNK, _CHUNK)]],
                g_v.at[pl.ds(j * _CHUNK, _CHUNK)], sem_g))

        def fetch(i, carry):
            t0 = i * _D
            idx16 = idx_v[pl.ds(t0, _D)]
            for k in range(_D):
                pltpu.async_copy(emb_hbm.at[idx16[k]], rows_v.at[t0 + k], sem_r)
            return carry

        lax.fori_loop(0, per_w // _D, fetch, 0)
        pltpu.make_async_copy(emb_hbm.at[pl.ds(0, per_w)], rows_v, sem_r).wait()
        for c in g_copies:
            c.wait()

        iota = lax.convert_element_type(lax.iota(jnp.int32, _D), jnp.float32)

        def step(i, carry):
            t0 = i * _D
            g16 = g_v[pl.ds(t0, _D)]
            for k in range(_D):
                rows_v[t0 + k] = jnp.where(iota < g16[k], rows_v[t0 + k], 0.0)
            return carry

        lax.fori_loop(0, per_w // _D, step, 0)
        pltpu.sync_copy(rows_v, out_hbm.at[pl.ds(base, per_w)])

    return body


def kernel(input, embedding, gates, index_array):
    b, l = input.shape
    n_tok = b * l
    idx = input.reshape(n_tok)
    out = _build(n_tok)(idx, embedding, gates)
    return out.reshape(b, l, _D)


# speedup vs baseline: 1.0509x; 1.0509x over previous
import functools

import jax
import jax.numpy as jnp
from jax import lax
from jax.experimental import pallas as pl
from jax.experimental.pallas import tpu as pltpu
from jax.experimental.pallas import tpu_sc as plsc

_D = 16
_NW = 32
_CHUNK = 128


@functools.cache
def _build(n_tok: int, b: int, l: int):
    per_w = n_tok // _NW
    n_ch = per_w // _CHUNK
    b_per_w = b // _NW
    mesh = plsc.VectorSubcoreMesh(core_axis_name="c", subcore_axis_name="s")

    @functools.partial(
        pl.kernel,
        mesh=mesh,
        out_type=jax.ShapeDtypeStruct((b, l, _D), jnp.float32),
        scratch_types=[
            pltpu.VMEM((per_w,), jnp.int32),
            pltpu.VMEM((per_w, _D), jnp.float32),
            pltpu.VMEM((per_w,), jnp.float32),
            pltpu.SemaphoreType.DMA,
            pltpu.SemaphoreType.DMA,
        ],
    )
    def body(idx_hbm, emb_hbm, gates_hbm, out_hbm, idx_v, rows_v, g_v, sem_r, sem_g):
        wid = lax.axis_index("s") * 2 + lax.axis_index("c")
        base = wid * per_w
        pltpu.sync_copy(idx_hbm.at[pl.ds(base, per_w)], idx_v)

        g_copies = []
        for j in range(n_ch):
            g_copies.append(pltpu.async_copy(
                gates_hbm.at[idx_v.at[pl.ds(j * _CHUNK, _CHUNK)]],
                g_v.at[pl.ds(j * _CHUNK, _CHUNK)], sem_g))

        def fetch(i, carry):
            t0 = i * _D
            idx16 = idx_v[pl.ds(t0, _D)]
            for k in range(_D):
                pltpu.async_copy(emb_hbm.at[idx16[k]], rows_v.at[t0 + k], sem_r)
            return carry

        lax.fori_loop(0, per_w // _D, fetch, 0)
        pltpu.make_async_copy(emb_hbm.at[pl.ds(0, per_w)], rows_v, sem_r).wait()
        for c in g_copies:
            c.wait()

        iota = lax.convert_element_type(lax.iota(jnp.int32, _D), jnp.float32)

        def step(i, carry):
            t0 = i * _D
            g16 = g_v[pl.ds(t0, _D)]
            for k in range(_D):
                rows_v[t0 + k] = jnp.where(iota < g16[k], rows_v[t0 + k], 0.0)
            return carry

        lax.fori_loop(0, per_w // _D, step, 0)
        out_copies = []
        for q in range(b_per_w):
            out_copies.append(pltpu.async_copy(
                rows_v.at[pl.ds(q * l, l)],
                out_hbm.at[wid * b_per_w + q], sem_r))
        for c in out_copies:
            c.wait()

    return body


def kernel(input, embedding, gates, index_array):
    b, l = input.shape
    n_tok = b * l
    idx = input.reshape(n_tok)
    return _build(n_tok, b, l)(idx, embedding, gates)
